# Initial kernel scaffold; baseline (speedup 1.0000x reference)
import jax
import jax.numpy as jnp
from jax.experimental import pallas as pl


def kernel(x, edge_index, W1, a_src1, a_dst1, b1, W2, a_src2, a_dst2, b2):
    raise NotImplementedError("write your pallas kernel here")


# speedup vs baseline: 16.2761x; 16.2761x over previous
import functools

import jax
import jax.numpy as jnp
from jax import lax
from jax.experimental import pallas as pl
from jax.experimental.pallas import tpu as pltpu
from jax.experimental.pallas import tpu_sc as plsc

TW = 144
BN = 400
NC = 2
NS = 16
CH = 80


def _proj1_body(x_ref, w_ref, as_ref, ad_ref, tab_ref, al_ref):
    n = x_ref.shape[0]
    m = jnp.dot(x_ref[...], w_ref[0], preferred_element_type=jnp.float32)
    tab_ref[0, :, 0:128] = m
    col = lax.broadcasted_iota(jnp.int32, (n, 16), 1)
    tab_ref[0, :, 128:144] = jnp.where(col == 0, 1.0, 0.0).astype(jnp.float32)
    al_ref[0, 0:1, :] = jnp.sum(m * as_ref[0], axis=1)[None, :]
    al_ref[0, 1:2, :] = jnp.sum(m * ad_ref[0], axis=1)[None, :]


def _tc_proj1(x, w1r, a_s, a_d, n, h, f):
    return pl.pallas_call(
        _proj1_body,
        grid=(h,),
        in_specs=[
            pl.BlockSpec((n, f), lambda j: (0, 0)),
            pl.BlockSpec((1, f, 128), lambda j: (j, 0, 0)),
            pl.BlockSpec((1, 1, 128), lambda j: (j, 0, 0)),
            pl.BlockSpec((1, 1, 128), lambda j: (j, 0, 0)),
        ],
        out_specs=[
            pl.BlockSpec((1, n, TW), lambda j: (j, 0, 0)),
            pl.BlockSpec((1, 2, n), lambda j: (j, 0, 0)),
        ],
        out_shape=[
            jax.ShapeDtypeStruct((h, n, TW), jnp.float32),
            jax.ShapeDtypeStruct((h, 2, n), jnp.float32),
        ],
    )(x, w1r, a_s, a_d)


def _mix_body(acc_ref, b1_ref, w2_ref, tab_ref):
    m2 = jnp.zeros((BN, 128), jnp.float32)
    for h in range(8):
        a = acc_ref[0, h] + acc_ref[1, h]
        hv = a[:, 0:128] / (a[:, 128:129] + 1e-16) + b1_ref[h, :][None, :]
        hv = jnp.where(hv > 0, hv, jnp.exp(hv) - 1.0)
        m2 = m2 + jnp.dot(hv, w2_ref[h], preferred_element_type=jnp.float32)
    tab_ref[:, 0:128] = m2
    col = lax.broadcasted_iota(jnp.int32, (BN, 16), 1)
    tab_ref[:, 128:144] = jnp.where(col == 0, 1.0, 0.0).astype(jnp.float32)


def _tc_mix(acc1, b1r, w2r, n):
    return pl.pallas_call(
        _mix_body,
        grid=(n // BN,),
        in_specs=[
            pl.BlockSpec((2, 8, BN, TW), lambda i: (0, 0, i, 0)),
            pl.BlockSpec((8, 128), lambda i: (0, 0)),
            pl.BlockSpec((8, 128, 128), lambda i: (0, 0, 0)),
        ],
        out_specs=pl.BlockSpec((BN, TW), lambda i: (i, 0)),
        out_shape=jax.ShapeDtypeStruct((n, TW), jnp.float32),
    )(acc1, b1r, w2r)


def _alpha2_body(tab_ref, as2_ref, ad2_ref, al_ref):
    t = tab_ref[:, 0:128]
    al_ref[0, 0:1, :] = jnp.sum(t * as2_ref[0], axis=1)[None, :]
    al_ref[0, 1:2, :] = jnp.sum(t * ad2_ref[0], axis=1)[None, :]


def _tc_alpha2(tab2, a_s2, a_d2, n):
    return pl.pallas_call(
        _alpha2_body,
        in_specs=[
            pl.BlockSpec((n, TW), lambda: (0, 0)),
            pl.BlockSpec((1, 1, 128), lambda: (0, 0, 0)),
            pl.BlockSpec((1, 1, 128), lambda: (0, 0, 0)),
        ],
        out_specs=pl.BlockSpec((1, 2, n), lambda: (0, 0, 0)),
        out_shape=jax.ShapeDtypeStruct((1, 2, n), jnp.float32),
    )(tab2, a_s2, a_d2)


def _final_body(acc_ref, b2_ref, out_ref):
    a = acc_ref[0, 0] + acc_ref[1, 0]
    out_ref[...] = a[:, 0:128] / (a[:, 128:129] + 1e-16) + b2_ref[0, :][None, :]


def _tc_final(acc2, b2r, n):
    return pl.pallas_call(
        _final_body,
        grid=(n // BN,),
        in_specs=[
            pl.BlockSpec((2, 1, BN, TW), lambda i: (0, 0, i, 0)),
            pl.BlockSpec((1, 128), lambda i: (0, 0)),
        ],
        out_specs=pl.BlockSpec((BN, 128), lambda i: (i, 0)),
        out_shape=jax.ShapeDtypeStruct((n, 128), jnp.float32),
    )(acc2, b2r)


def _make_sc_edge(num_heads, n, e):
    ept = e // (NC * NS)
    sce = 2000
    nsc = ept // sce
    nch = sce // CH
    rpt = 624
    mesh = plsc.VectorSubcoreMesh(core_axis_name="c", subcore_axis_name="s")

    @functools.partial(
        pl.kernel,
        mesh=mesh,
        compiler_params=pltpu.CompilerParams(needs_layout_passes=False,
                                             use_tc_tiling_on_sc=False),
        out_type=jax.ShapeDtypeStruct((NC, num_heads, n, TW), jnp.float32),
        scratch_types=[
            pltpu.VMEM((sce,), jnp.int32),
            pltpu.VMEM((sce,), jnp.int32),
            pltpu.VMEM((n,), jnp.float32),
            pltpu.VMEM((n,), jnp.float32),
            pltpu.VMEM((CH,), jnp.int32),
            pltpu.VMEM((CH,), jnp.int32),
            pltpu.VMEM((CH,), jnp.float32),
            pltpu.VMEM((CH, TW), jnp.float32),
            pltpu.VMEM((16, TW), jnp.float32),
            pltpu.VMEM_SHARED((n, TW), jnp.float32),
            pltpu.SemaphoreType.DMA,
        ],
    )
    def sck(tab_hbm, al_hbm, srce_hbm, dste_hbm, out_hbm,
            src_t, dst_t, as_t, ad_t, gix_t, dix_t, w_t, rows_t, zero_t,
            acc_sh, sem):
        sid = lax.axis_index("s")
        cid = lax.axis_index("c")
        wid = sid * NC + cid
        base = wid * ept
        z16 = jnp.zeros((16,), jnp.float32)

        def _fill_zero(k, _):
            i = k // (TW // 16)
            j = k % (TW // 16)
            zero_t[i, pl.ds(j * 16, 16)] = z16
            return _

        lax.fori_loop(0, 16 * (TW // 16), _fill_zero, None)

        nz = jnp.where(sid == NS - 1, 40, 39)

        def _clear_acc():
            def _z(z, _):
                pltpu.sync_copy(
                    zero_t, acc_sh.at[pl.ds(sid * rpt + z * 16, 16), :])
                return _
            lax.fori_loop(0, nz, _z, None)

        _clear_acc()
        plsc.subcore_barrier()

        def _head(h, _):
            pltpu.sync_copy(al_hbm.at[h, 0], as_t)
            pltpu.sync_copy(al_hbm.at[h, 1], ad_t)
            hbase = h * n

            def _sup(sc_i, _):
                pltpu.sync_copy(
                    srce_hbm.at[pl.ds(base + sc_i * sce, sce)], src_t)
                pltpu.sync_copy(
                    dste_hbm.at[pl.ds(base + sc_i * sce, sce)], dst_t)
                lax.fori_loop(0, nch, _chunk, None)
                return _

            def _chunk(c, _):
                eoff = c * CH

                def _wgrp(g, _):
                    s16 = src_t[pl.ds(eoff + g * 16, 16)]
                    d16 = dst_t[pl.ds(eoff + g * 16, 16)]
                    gix_t[pl.ds(g * 16, 16)] = s16 + hbase
                    dix_t[pl.ds(g * 16, 16)] = d16
                    ev = (plsc.load_gather(as_t, [s16])
                          + plsc.load_gather(ad_t, [d16]))
                    ev = jnp.where(ev > 0, ev, 0.2 * ev)
                    w_t[pl.ds(g * 16, 16)] = jnp.exp(ev)
                    return _

                lax.fori_loop(0, CH // 16, _wgrp, None)
                pltpu.async_copy(tab_hbm.at[gix_t], rows_t, sem).wait()

                def _scale(g, _):
                    w16 = w_t[pl.ds(g * 16, 16)]
                    for lane in range(16):
                        ws = w16[lane]
                        ei = g * 16 + lane
                        for s in range(TW // 16):
                            rows_t[ei, pl.ds(s * 16, 16)] = (
                                rows_t[ei, pl.ds(s * 16, 16)] * ws)
                    return _

                lax.fori_loop(0, CH // 16, _scale, None)
                pltpu.sync_copy(rows_t, acc_sh.at[dix_t], add=True)
                return _

            lax.fori_loop(0, nsc, _sup, None)
            plsc.subcore_barrier()
            pltpu.sync_copy(acc_sh.at[pl.ds(sid * rpt, rpt), :],
                            out_hbm.at[cid, h, pl.ds(sid * rpt, rpt), :])

            @pl.when(sid == NS - 1)
            def _():
                pltpu.sync_copy(
                    acc_sh.at[pl.ds(sid * rpt + rpt, 16), :],
                    out_hbm.at[cid, h, pl.ds(sid * rpt + rpt, 16), :])

            _clear_acc()
            plsc.subcore_barrier()
            return _

        lax.fori_loop(0, num_heads, _head, None)

    return sck


def kernel(x, edge_index, W1, a_src1, a_dst1, b1, W2, a_src2, a_dst2, b2):
    n, f = x.shape
    e = edge_index.shape[1]
    h1 = a_src1.shape[0]

    src = edge_index[0]
    dst = edge_index[1]
    w1r = W1.reshape(f, h1, 128).transpose(1, 0, 2)
    tab1, al1 = _tc_proj1(x, w1r, a_src1.reshape(h1, 1, 128),
                          a_dst1.reshape(h1, 1, 128), n, h1, f)
    acc1 = _make_sc_edge(h1, n, e)(tab1.reshape(h1 * n, TW), al1, src, dst)
    tab2 = _tc_mix(acc1, b1.reshape(h1, 128), W2.reshape(h1, 128, 128), n)
    al2 = _tc_alpha2(tab2, a_src2.reshape(1, 1, 128),
                     a_dst2.reshape(1, 1, 128), n)
    acc2 = _make_sc_edge(1, n, e)(tab2, al2, src, dst)
    return _tc_final(acc2, b2.reshape(1, 128), n)


# speedup vs baseline: 24.3038x; 1.4932x over previous
import functools

import jax
import jax.numpy as jnp
from jax import lax
from jax.experimental import pallas as pl
from jax.experimental.pallas import tpu as pltpu
from jax.experimental.pallas import tpu_sc as plsc

TW = 144
BN = 400
NC = 2
NS = 16
CH = 80


def _proj1_body(x_ref, w_ref, as_ref, ad_ref, tab_ref, al_ref):
    n = x_ref.shape[0]
    m = jnp.dot(x_ref[...], w_ref[0], preferred_element_type=jnp.float32)
    tab_ref[0, :, 0:128] = m
    col = lax.broadcasted_iota(jnp.int32, (n, 16), 1)
    tab_ref[0, :, 128:144] = jnp.where(col == 0, 1.0, 0.0).astype(jnp.float32)
    al_ref[0, 0:1, :] = jnp.sum(m * as_ref[0], axis=1)[None, :]
    al_ref[0, 1:2, :] = jnp.sum(m * ad_ref[0], axis=1)[None, :]


def _tc_proj1(x, w1r, a_s, a_d, n, h, f):
    return pl.pallas_call(
        _proj1_body,
        grid=(h,),
        in_specs=[
            pl.BlockSpec((n, f), lambda j: (0, 0)),
            pl.BlockSpec((1, f, 128), lambda j: (j, 0, 0)),
            pl.BlockSpec((1, 1, 128), lambda j: (j, 0, 0)),
            pl.BlockSpec((1, 1, 128), lambda j: (j, 0, 0)),
        ],
        out_specs=[
            pl.BlockSpec((1, n, TW), lambda j: (j, 0, 0)),
            pl.BlockSpec((1, 2, n), lambda j: (j, 0, 0)),
        ],
        out_shape=[
            jax.ShapeDtypeStruct((h, n, TW), jnp.float32),
            jax.ShapeDtypeStruct((h, 2, n), jnp.float32),
        ],
    )(x, w1r, a_s, a_d)


def _mix_body(acc_ref, b1_ref, w2_ref, tab_ref):
    m2 = jnp.zeros((BN, 128), jnp.float32)
    for h in range(8):
        a = acc_ref[0, h] + acc_ref[1, h]
        hv = a[:, 0:128] / (a[:, 128:129] + 1e-16) + b1_ref[h, :][None, :]
        hv = jnp.where(hv > 0, hv, jnp.exp(hv) - 1.0)
        m2 = m2 + jnp.dot(hv, w2_ref[h], preferred_element_type=jnp.float32)
    tab_ref[:, 0:128] = m2
    col = lax.broadcasted_iota(jnp.int32, (BN, 16), 1)
    tab_ref[:, 128:144] = jnp.where(col == 0, 1.0, 0.0).astype(jnp.float32)


def _tc_mix(acc1, b1r, w2r, n):
    return pl.pallas_call(
        _mix_body,
        grid=(n // BN,),
        in_specs=[
            pl.BlockSpec((2, 8, BN, TW), lambda i: (0, 0, i, 0)),
            pl.BlockSpec((8, 128), lambda i: (0, 0)),
            pl.BlockSpec((8, 128, 128), lambda i: (0, 0, 0)),
        ],
        out_specs=pl.BlockSpec((BN, TW), lambda i: (i, 0)),
        out_shape=jax.ShapeDtypeStruct((n, TW), jnp.float32),
    )(acc1, b1r, w2r)


def _alpha2_body(tab_ref, as2_ref, ad2_ref, al_ref):
    t = tab_ref[:, 0:128]
    al_ref[0, 0:1, :] = jnp.sum(t * as2_ref[0], axis=1)[None, :]
    al_ref[0, 1:2, :] = jnp.sum(t * ad2_ref[0], axis=1)[None, :]


def _tc_alpha2(tab2, a_s2, a_d2, n):
    return pl.pallas_call(
        _alpha2_body,
        in_specs=[
            pl.BlockSpec((n, TW), lambda: (0, 0)),
            pl.BlockSpec((1, 1, 128), lambda: (0, 0, 0)),
            pl.BlockSpec((1, 1, 128), lambda: (0, 0, 0)),
        ],
        out_specs=pl.BlockSpec((1, 2, n), lambda: (0, 0, 0)),
        out_shape=jax.ShapeDtypeStruct((1, 2, n), jnp.float32),
    )(tab2, a_s2, a_d2)


def _final_body(acc_ref, b2_ref, out_ref):
    a = acc_ref[0, 0] + acc_ref[1, 0]
    out_ref[...] = a[:, 0:128] / (a[:, 128:129] + 1e-16) + b2_ref[0, :][None, :]


def _tc_final(acc2, b2r, n):
    return pl.pallas_call(
        _final_body,
        grid=(n // BN,),
        in_specs=[
            pl.BlockSpec((2, 1, BN, TW), lambda i: (0, 0, i, 0)),
            pl.BlockSpec((1, 128), lambda i: (0, 0)),
        ],
        out_specs=pl.BlockSpec((BN, 128), lambda i: (i, 0)),
        out_shape=jax.ShapeDtypeStruct((n, 128), jnp.float32),
    )(acc2, b2r)


def _make_sc_edge(num_heads, n, e):
    ept = e // (NC * NS)
    sce = 2000
    nsc = ept // sce
    nch = sce // CH
    rpt = 624
    mesh = plsc.VectorSubcoreMesh(core_axis_name="c", subcore_axis_name="s")

    @functools.partial(
        pl.kernel,
        mesh=mesh,
        compiler_params=pltpu.CompilerParams(needs_layout_passes=False,
                                             use_tc_tiling_on_sc=False),
        out_type=jax.ShapeDtypeStruct((NC, num_heads, n, TW), jnp.float32),
        scratch_types=[
            pltpu.VMEM((sce,), jnp.int32),
            pltpu.VMEM((sce,), jnp.int32),
            pltpu.VMEM((CH,), jnp.int32),
            pltpu.VMEM((CH,), jnp.int32),
            pltpu.VMEM((CH,), jnp.int32),
            pltpu.VMEM((CH,), jnp.int32),
            pltpu.VMEM((CH,), jnp.int32),
            pltpu.VMEM((CH,), jnp.int32),
            pltpu.VMEM((CH,), jnp.float32),
            pltpu.VMEM((CH,), jnp.float32),
            pltpu.VMEM((CH,), jnp.float32),
            pltpu.VMEM((CH,), jnp.float32),
            pltpu.VMEM((CH, TW), jnp.float32),
            pltpu.VMEM((CH, TW), jnp.float32),
            pltpu.VMEM((16, TW), jnp.float32),
            pltpu.VMEM_SHARED((n, TW), jnp.float32),
            pltpu.SemaphoreType.DMA,
            pltpu.SemaphoreType.DMA,
            pltpu.SemaphoreType.DMA,
            pltpu.SemaphoreType.DMA,
        ],
    )
    def sck(tab_hbm, als_hbm, ald_hbm, srce_hbm, dste_hbm, out_hbm,
            src_t, dst_t, gix0, gix1, dix0, dix1, aix0, aix1,
            asb0, asb1, adb0, adb1, rows0, rows1, zero_t,
            acc_sh, gsem0, gsem1, ssem0, ssem1):
        sid = lax.axis_index("s")
        cid = lax.axis_index("c")
        wid = sid * NC + cid
        base = wid * ept
        z16 = jnp.zeros((16,), jnp.float32)
        buf = ((gix0, dix0, aix0, asb0, adb0, rows0, gsem0, ssem0),
               (gix1, dix1, aix1, asb1, adb1, rows1, gsem1, ssem1))

        def _fill_zero(k, _):
            i = k // (TW // 16)
            j = k % (TW // 16)
            zero_t[i, pl.ds(j * 16, 16)] = z16
            return _

        lax.fori_loop(0, 16 * (TW // 16), _fill_zero, None)

        nz = jnp.where(sid == NS - 1, 40, 39)

        def _clear_acc():
            def _z(z, _):
                pltpu.sync_copy(
                    zero_t, acc_sh.at[pl.ds(sid * rpt + z * 16, 16), :])
                return _
            lax.fori_loop(0, nz, _z, None)

        _clear_acc()
        plsc.subcore_barrier()

        def _head(h, _):
            hbase = h * n

            def _do_idx(coff, b):
                gix, dix, aix = buf[b][0], buf[b][1], buf[b][2]

                def _ix(g, _):
                    s16 = src_t[pl.ds(coff + g * 16, 16)]
                    d16 = dst_t[pl.ds(coff + g * 16, 16)]
                    gix[pl.ds(g * 16, 16)] = s16 + hbase
                    dix[pl.ds(g * 16, 16)] = d16
                    aix[pl.ds(g * 16, 16)] = d16 + hbase
                    return _

                lax.fori_loop(0, CH // 16, _ix, None)

            def _issue_gathers(b):
                gix, dix, aix, asb, adb, rows, gsem, _s = buf[b]
                pltpu.async_copy(tab_hbm.at[gix], rows, gsem)
                pltpu.async_copy(als_hbm.at[gix], asb, gsem)
                pltpu.async_copy(ald_hbm.at[aix], adb, gsem)

            def _wait_gathers(b):
                gix, dix, aix, asb, adb, rows, gsem, _s = buf[b]
                pltpu.make_async_copy(tab_hbm.at[gix], rows, gsem).wait()
                pltpu.make_async_copy(als_hbm.at[gix], asb, gsem).wait()
                pltpu.make_async_copy(ald_hbm.at[aix], adb, gsem).wait()

            def _scatter(b):
                rows, ssem = buf[b][5], buf[b][7]
                dix = buf[b][1]
                pltpu.async_copy(rows, acc_sh.at[dix], ssem, add=True)

            def _wait_scatter(b):
                rows, ssem = buf[b][5], buf[b][7]
                dix = buf[b][1]
                pltpu.make_async_copy(rows, acc_sh.at[dix], ssem).wait()

            def _process(b):
                asb, adb, rows = buf[b][3], buf[b][4], buf[b][5]

                def _grp(g, _):
                    ev = asb[pl.ds(g * 16, 16)] + adb[pl.ds(g * 16, 16)]
                    ev = jnp.where(ev > 0, ev, 0.2 * ev)
                    w16 = jnp.exp(ev)
                    for lane in range(16):
                        ws = w16[lane]
                        ei = g * 16 + lane
                        for s in range(TW // 16):
                            rows[ei, pl.ds(s * 16, 16)] = (
                                rows[ei, pl.ds(s * 16, 16)] * ws)
                    return _

                lax.fori_loop(0, CH // 16, _grp, None)

            def _sup(sc_i, _):
                pltpu.sync_copy(
                    srce_hbm.at[pl.ds(base + sc_i * sce, sce)], src_t)
                pltpu.sync_copy(
                    dste_hbm.at[pl.ds(base + sc_i * sce, sce)], dst_t)
                _do_idx(0, 0)
                _issue_gathers(0)

                def _pair(i, _):
                    coff = i * 2 * CH

                    @pl.when(i > 0)
                    def _():
                        _wait_scatter(1)
                    _do_idx(coff + CH, 1)
                    _issue_gathers(1)
                    _wait_gathers(0)
                    _process(0)
                    _scatter(0)
                    _wait_scatter(0)
                    _do_idx(coff + 2 * CH, 0)
                    _issue_gathers(0)
                    _wait_gathers(1)
                    _process(1)
                    _scatter(1)
                    return _

                lax.fori_loop(0, (nch - 1) // 2, _pair, None)
                _wait_scatter(1)
                _wait_gathers(0)
                _process(0)
                _scatter(0)
                _wait_scatter(0)
                return _

            lax.fori_loop(0, nsc, _sup, None)
            plsc.subcore_barrier()
            pltpu.sync_copy(acc_sh.at[pl.ds(sid * rpt, rpt), :],
                            out_hbm.at[cid, h, pl.ds(sid * rpt, rpt), :])

            @pl.when(sid == NS - 1)
            def _():
                pltpu.sync_copy(
                    acc_sh.at[pl.ds(sid * rpt + rpt, 16), :],
                    out_hbm.at[cid, h, pl.ds(sid * rpt + rpt, 16), :])

            _clear_acc()
            plsc.subcore_barrier()
            return _

        lax.fori_loop(0, num_heads, _head, None)

    return sck


def kernel(x, edge_index, W1, a_src1, a_dst1, b1, W2, a_src2, a_dst2, b2):
    n, f = x.shape
    e = edge_index.shape[1]
    h1 = a_src1.shape[0]

    src = edge_index[0]
    dst = edge_index[1]
    w1r = W1.reshape(f, h1, 128).transpose(1, 0, 2)
    tab1, al1 = _tc_proj1(x, w1r, a_src1.reshape(h1, 1, 128),
                          a_dst1.reshape(h1, 1, 128), n, h1, f)
    acc1 = _make_sc_edge(h1, n, e)(tab1.reshape(h1 * n, TW),
                                   al1[:, 0, :].reshape(h1 * n),
                                   al1[:, 1, :].reshape(h1 * n), src, dst)
    tab2 = _tc_mix(acc1, b1.reshape(h1, 128), W2.reshape(h1, 128, 128), n)
    al2 = _tc_alpha2(tab2, a_src2.reshape(1, 1, 128),
                     a_dst2.reshape(1, 1, 128), n)
    acc2 = _make_sc_edge(1, n, e)(tab2, al2[:, 0, :].reshape(n),
                                  al2[:, 1, :].reshape(n), src, dst)
    return _tc_final(acc2, b2.reshape(1, 128), n)


# speedup vs baseline: 26.5844x; 1.0938x over previous
import functools

import jax
import jax.numpy as jnp
from jax import lax
from jax.experimental import pallas as pl
from jax.experimental.pallas import tpu as pltpu
from jax.experimental.pallas import tpu_sc as plsc

TW = 144
BN = 400
NC = 2
NS = 16
CH = 80


def _proj1_body(x_ref, w_ref, as_ref, ad_ref, tab_ref, al_ref):
    n = x_ref.shape[0]
    m = jnp.dot(x_ref[...], w_ref[0], preferred_element_type=jnp.float32)
    tab_ref[0, :, 0:128] = m
    col = lax.broadcasted_iota(jnp.int32, (n, 16), 1)
    tab_ref[0, :, 128:144] = jnp.where(col == 0, 1.0, 0.0).astype(jnp.float32)
    al_ref[0, 0:1, :] = jnp.sum(m * as_ref[0], axis=1)[None, :]
    al_ref[0, 1:2, :] = jnp.sum(m * ad_ref[0], axis=1)[None, :]


def _tc_proj1(x, w1r, a_s, a_d, n, h, f):
    return pl.pallas_call(
        _proj1_body,
        grid=(h,),
        in_specs=[
            pl.BlockSpec((n, f), lambda j: (0, 0)),
            pl.BlockSpec((1, f, 128), lambda j: (j, 0, 0)),
            pl.BlockSpec((1, 1, 128), lambda j: (j, 0, 0)),
            pl.BlockSpec((1, 1, 128), lambda j: (j, 0, 0)),
        ],
        out_specs=[
            pl.BlockSpec((1, n, TW), lambda j: (j, 0, 0)),
            pl.BlockSpec((1, 2, n), lambda j: (j, 0, 0)),
        ],
        out_shape=[
            jax.ShapeDtypeStruct((h, n, TW), jnp.float32),
            jax.ShapeDtypeStruct((h, 2, n), jnp.float32),
        ],
    )(x, w1r, a_s, a_d)


def _mix_body(acc_ref, b1_ref, w2_ref, tab_ref):
    m2 = jnp.zeros((BN, 128), jnp.float32)
    for h in range(8):
        a = acc_ref[0, h] + acc_ref[1, h]
        hv = a[:, 0:128] / (a[:, 128:129] + 1e-16) + b1_ref[h, :][None, :]
        hv = jnp.where(hv > 0, hv, jnp.exp(hv) - 1.0)
        m2 = m2 + jnp.dot(hv, w2_ref[h], preferred_element_type=jnp.float32)
    tab_ref[:, 0:128] = m2
    col = lax.broadcasted_iota(jnp.int32, (BN, 16), 1)
    tab_ref[:, 128:144] = jnp.where(col == 0, 1.0, 0.0).astype(jnp.float32)


def _tc_mix(acc1, b1r, w2r, n):
    return pl.pallas_call(
        _mix_body,
        grid=(n // BN,),
        in_specs=[
            pl.BlockSpec((2, 8, BN, TW), lambda i: (0, 0, i, 0)),
            pl.BlockSpec((8, 128), lambda i: (0, 0)),
            pl.BlockSpec((8, 128, 128), lambda i: (0, 0, 0)),
        ],
        out_specs=pl.BlockSpec((BN, TW), lambda i: (i, 0)),
        out_shape=jax.ShapeDtypeStruct((n, TW), jnp.float32),
    )(acc1, b1r, w2r)


def _alpha2_body(tab_ref, as2_ref, ad2_ref, al_ref):
    t = tab_ref[:, 0:128]
    al_ref[0, 0:1, :] = jnp.sum(t * as2_ref[0], axis=1)[None, :]
    al_ref[0, 1:2, :] = jnp.sum(t * ad2_ref[0], axis=1)[None, :]


def _tc_alpha2(tab2, a_s2, a_d2, n):
    return pl.pallas_call(
        _alpha2_body,
        in_specs=[
            pl.BlockSpec((n, TW), lambda: (0, 0)),
            pl.BlockSpec((1, 1, 128), lambda: (0, 0, 0)),
            pl.BlockSpec((1, 1, 128), lambda: (0, 0, 0)),
        ],
        out_specs=pl.BlockSpec((1, 2, n), lambda: (0, 0, 0)),
        out_shape=jax.ShapeDtypeStruct((1, 2, n), jnp.float32),
    )(tab2, a_s2, a_d2)


def _final_body(acc_ref, b2_ref, out_ref):
    a = acc_ref[0, 0] + acc_ref[1, 0]
    out_ref[...] = a[:, 0:128] / (a[:, 128:129] + 1e-16) + b2_ref[0, :][None, :]


def _tc_final(acc2, b2r, n):
    return pl.pallas_call(
        _final_body,
        grid=(n // BN,),
        in_specs=[
            pl.BlockSpec((2, 1, BN, TW), lambda i: (0, 0, i, 0)),
            pl.BlockSpec((1, 128), lambda i: (0, 0)),
        ],
        out_specs=pl.BlockSpec((BN, 128), lambda i: (i, 0)),
        out_shape=jax.ShapeDtypeStruct((n, 128), jnp.float32),
    )(acc2, b2r)


def _make_sc_edge(num_heads, n, e):
    ept = e // (NC * NS)
    sce = 2000
    nsc = ept // sce
    nch = sce // CH
    rpt = 624
    mesh = plsc.VectorSubcoreMesh(core_axis_name="c", subcore_axis_name="s")

    @functools.partial(
        pl.kernel,
        mesh=mesh,
        compiler_params=pltpu.CompilerParams(needs_layout_passes=False,
                                             use_tc_tiling_on_sc=False),
        out_type=jax.ShapeDtypeStruct((NC, num_heads, n, TW), jnp.float32),
        scratch_types=[
            pltpu.VMEM((sce,), jnp.int32),
            pltpu.VMEM((sce,), jnp.int32),
            pltpu.VMEM((CH,), jnp.int32),
            pltpu.VMEM((CH,), jnp.int32),
            pltpu.VMEM((CH,), jnp.int32),
            pltpu.VMEM((CH,), jnp.int32),
            pltpu.VMEM((CH,), jnp.int32),
            pltpu.VMEM((CH,), jnp.int32),
            pltpu.VMEM((CH,), jnp.int32),
            pltpu.VMEM((CH,), jnp.int32),
            pltpu.VMEM((CH,), jnp.int32),
            pltpu.VMEM((CH,), jnp.float32),
            pltpu.VMEM((CH,), jnp.float32),
            pltpu.VMEM((CH,), jnp.float32),
            pltpu.VMEM((CH,), jnp.float32),
            pltpu.VMEM((CH,), jnp.float32),
            pltpu.VMEM((CH,), jnp.float32),
            pltpu.VMEM((CH, TW), jnp.float32),
            pltpu.VMEM((CH, TW), jnp.float32),
            pltpu.VMEM((CH, TW), jnp.float32),
            pltpu.VMEM_SHARED((n, TW), jnp.float32),
            pltpu.SemaphoreType.DMA,
            pltpu.SemaphoreType.DMA,
            pltpu.SemaphoreType.DMA,
            pltpu.SemaphoreType.DMA,
            pltpu.SemaphoreType.DMA,
            pltpu.SemaphoreType.DMA,
        ],
    )
    def sck(tab_hbm, als_hbm, ald_hbm, srce_hbm, dste_hbm, out_hbm,
            src_t, dst_t, gix0, gix1, gix2, dix0, dix1, dix2,
            aix0, aix1, aix2, asb0, asb1, asb2, adb0, adb1, adb2,
            rows0, rows1, rows2,
            acc_sh, gsem0, gsem1, gsem2, ssem0, ssem1, ssem2):
        sid = lax.axis_index("s")
        cid = lax.axis_index("c")
        wid = sid * NC + cid
        base = wid * ept
        z16 = jnp.zeros((16,), jnp.float32)
        buf = ((gix0, dix0, aix0, asb0, adb0, rows0, gsem0, ssem0),
               (gix1, dix1, aix1, asb1, adb1, rows1, gsem1, ssem1),
               (gix2, dix2, aix2, asb2, adb2, rows2, gsem2, ssem2))

        def _fill_zero(k, _):
            i = k // (TW // 16)
            j = k % (TW // 16)
            rows0[i, pl.ds(j * 16, 16)] = z16
            return _

        nz = jnp.where(sid == NS - 1, 40, 39)

        def _clear_acc():
            lax.fori_loop(0, 16 * (TW // 16), _fill_zero, None)

            def _z(z, _):
                pltpu.sync_copy(
                    rows0.at[pl.ds(0, 16), :],
                    acc_sh.at[pl.ds(sid * rpt + z * 16, 16), :])
                return _
            lax.fori_loop(0, nz, _z, None)

        _clear_acc()
        plsc.subcore_barrier()

        def _head(h, _):
            hbase = h * n

            def _do_idx(coff, b):
                gix, dix, aix = buf[b][0], buf[b][1], buf[b][2]

                def _ix(g, _):
                    s16 = src_t[pl.ds(coff + g * 16, 16)]
                    d16 = dst_t[pl.ds(coff + g * 16, 16)]
                    gix[pl.ds(g * 16, 16)] = s16 + hbase
                    dix[pl.ds(g * 16, 16)] = d16
                    aix[pl.ds(g * 16, 16)] = d16 + hbase
                    return _

                lax.fori_loop(0, CH // 16, _ix, None)

            def _issue_gathers(b):
                gix, dix, aix, asb, adb, rows, gsem, _s = buf[b]
                pltpu.async_copy(tab_hbm.at[gix], rows, gsem)
                pltpu.async_copy(als_hbm.at[gix], asb, gsem)
                pltpu.async_copy(ald_hbm.at[aix], adb, gsem)

            def _wait_gathers(b):
                gix, dix, aix, asb, adb, rows, gsem, _s = buf[b]
                pltpu.make_async_copy(tab_hbm.at[gix], rows, gsem).wait()
                pltpu.make_async_copy(als_hbm.at[gix], asb, gsem).wait()
                pltpu.make_async_copy(ald_hbm.at[aix], adb, gsem).wait()

            def _scatter(b):
                rows, ssem = buf[b][5], buf[b][7]
                dix = buf[b][1]
                pltpu.async_copy(rows, acc_sh.at[dix], ssem, add=True)

            def _wait_scatter(b):
                rows, ssem = buf[b][5], buf[b][7]
                dix = buf[b][1]
                pltpu.make_async_copy(rows, acc_sh.at[dix], ssem).wait()

            def _process(b):
                asb, adb, rows = buf[b][3], buf[b][4], buf[b][5]

                def _grp(g, _):
                    ev = asb[pl.ds(g * 16, 16)] + adb[pl.ds(g * 16, 16)]
                    ev = jnp.where(ev > 0, ev, 0.2 * ev)
                    w16 = jnp.exp(ev)
                    for lane in range(16):
                        ws = w16[lane]
                        ei = g * 16 + lane
                        for s in range(TW // 16):
                            rows[ei, pl.ds(s * 16, 16)] = (
                                rows[ei, pl.ds(s * 16, 16)] * ws)
                    return _

                lax.fori_loop(0, CH // 16, _grp, None)

            def _phase(coff, b, guard_first=False):
                bn = (b + 2) % 3
                _wait_gathers(b)
                _process(b)
                _scatter(b)
                if guard_first:
                    pass
                else:
                    _wait_scatter(bn)
                _do_idx(coff + 2 * CH, bn)
                _issue_gathers(bn)

            def _sup(sc_i, _):
                pltpu.sync_copy(
                    srce_hbm.at[pl.ds(base + sc_i * sce, sce)], src_t)
                pltpu.sync_copy(
                    dste_hbm.at[pl.ds(base + sc_i * sce, sce)], dst_t)
                _do_idx(0, 0)
                _issue_gathers(0)
                _do_idx(CH, 1)
                _issue_gathers(1)

                def _triple(i, _):
                    coff = i * 3 * CH

                    _wait_gathers(0)
                    _process(0)
                    _scatter(0)

                    @pl.when(i > 0)
                    def _():
                        _wait_scatter(2)
                    _do_idx(coff + 2 * CH, 2)
                    _issue_gathers(2)
                    _phase(coff + CH, 1)
                    _phase(coff + 2 * CH, 2)
                    return _

                lax.fori_loop(0, (nch - 4) // 3, _triple, None)
                c0 = (nch - 4) // 3 * 3 * CH
                _phase(c0, 0)
                _phase(c0 + CH, 1)
                _wait_gathers(2)
                _process(2)
                _scatter(2)
                _wait_scatter(1)
                _wait_gathers(0)
                _process(0)
                _scatter(0)
                _wait_scatter(2)
                _wait_scatter(0)
                return _

            lax.fori_loop(0, nsc, _sup, None)
            plsc.subcore_barrier()
            pltpu.sync_copy(acc_sh.at[pl.ds(sid * rpt, rpt), :],
                            out_hbm.at[cid, h, pl.ds(sid * rpt, rpt), :])

            @pl.when(sid == NS - 1)
            def _():
                pltpu.sync_copy(
                    acc_sh.at[pl.ds(sid * rpt + rpt, 16), :],
                    out_hbm.at[cid, h, pl.ds(sid * rpt + rpt, 16), :])

            _clear_acc()
            plsc.subcore_barrier()
            return _

        lax.fori_loop(0, num_heads, _head, None)

    return sck


def kernel(x, edge_index, W1, a_src1, a_dst1, b1, W2, a_src2, a_dst2, b2):
    n, f = x.shape
    e = edge_index.shape[1]
    h1 = a_src1.shape[0]

    src = edge_index[0]
    dst = edge_index[1]
    w1r = W1.reshape(f, h1, 128).transpose(1, 0, 2)
    tab1, al1 = _tc_proj1(x, w1r, a_src1.reshape(h1, 1, 128),
                          a_dst1.reshape(h1, 1, 128), n, h1, f)
    acc1 = _make_sc_edge(h1, n, e)(tab1.reshape(h1 * n, TW),
                                   al1[:, 0, :].reshape(h1 * n),
                                   al1[:, 1, :].reshape(h1 * n), src, dst)
    tab2 = _tc_mix(acc1, b1.reshape(h1, 128), W2.reshape(h1, 128, 128), n)
    al2 = _tc_alpha2(tab2, a_src2.reshape(1, 1, 128),
                     a_dst2.reshape(1, 1, 128), n)
    acc2 = _make_sc_edge(1, n, e)(tab2, al2[:, 0, :].reshape(n),
                                  al2[:, 1, :].reshape(n), src, dst)
    return _tc_final(acc2, b2.reshape(1, 128), n)


# speedup vs baseline: 26.8217x; 1.0089x over previous
import functools

import jax
import jax.numpy as jnp
from jax import lax
from jax.experimental import pallas as pl
from jax.experimental.pallas import tpu as pltpu
from jax.experimental.pallas import tpu_sc as plsc

TW = 144
BN = 400
NC = 2
NS = 16
CH = 80


def _proj1_body(x_ref, w_ref, as_ref, ad_ref, tab_ref, al_ref):
    n = x_ref.shape[0]
    m = jnp.dot(x_ref[...], w_ref[0], preferred_element_type=jnp.float32)
    tab_ref[0, :, 0:128] = m
    col = lax.broadcasted_iota(jnp.int32, (n, 16), 1)
    tab_ref[0, :, 128:144] = jnp.where(col == 0, 1.0, 0.0).astype(jnp.float32)
    al_ref[0, 0:1, :] = jnp.sum(m * as_ref[0], axis=1)[None, :]
    al_ref[0, 1:2, :] = jnp.sum(m * ad_ref[0], axis=1)[None, :]


def _tc_proj1(x, w1r, a_s, a_d, n, h, f):
    return pl.pallas_call(
        _proj1_body,
        grid=(h,),
        in_specs=[
            pl.BlockSpec((n, f), lambda j: (0, 0)),
            pl.BlockSpec((1, f, 128), lambda j: (j, 0, 0)),
            pl.BlockSpec((1, 1, 128), lambda j: (j, 0, 0)),
            pl.BlockSpec((1, 1, 128), lambda j: (j, 0, 0)),
        ],
        out_specs=[
            pl.BlockSpec((1, n, TW), lambda j: (j, 0, 0)),
            pl.BlockSpec((1, 2, n), lambda j: (j, 0, 0)),
        ],
        out_shape=[
            jax.ShapeDtypeStruct((h, n, TW), jnp.float32),
            jax.ShapeDtypeStruct((h, 2, n), jnp.float32),
        ],
    )(x, w1r, a_s, a_d)


def _mix_body(acc_ref, b1_ref, w2_ref, as2_ref, ad2_ref, tab_ref, al_ref):
    m2 = jnp.zeros((BN, 128), jnp.float32)
    for h in range(8):
        a = acc_ref[0, h] + acc_ref[1, h]
        hv = a[:, 0:128] / (a[:, 128:129] + 1e-16) + b1_ref[h, :][None, :]
        hv = jnp.where(hv > 0, hv, jnp.exp(hv) - 1.0)
        m2 = m2 + jnp.dot(hv, w2_ref[h], preferred_element_type=jnp.float32)
    tab_ref[:, 0:128] = m2
    col = lax.broadcasted_iota(jnp.int32, (BN, 16), 1)
    tab_ref[:, 128:144] = jnp.where(col == 0, 1.0, 0.0).astype(jnp.float32)
    al_ref[:, 0:1] = jnp.sum(m2 * as2_ref[0], axis=1)[:, None]
    al_ref[:, 1:2] = jnp.sum(m2 * ad2_ref[0], axis=1)[:, None]


def _tc_mix(acc1, b1r, w2r, a_s2, a_d2, n):
    return pl.pallas_call(
        _mix_body,
        grid=(n // BN,),
        in_specs=[
            pl.BlockSpec((2, 8, BN, TW), lambda i: (0, 0, i, 0)),
            pl.BlockSpec((8, 128), lambda i: (0, 0)),
            pl.BlockSpec((8, 128, 128), lambda i: (0, 0, 0)),
            pl.BlockSpec((1, 1, 128), lambda i: (0, 0, 0)),
            pl.BlockSpec((1, 1, 128), lambda i: (0, 0, 0)),
        ],
        out_specs=[
            pl.BlockSpec((BN, TW), lambda i: (i, 0)),
            pl.BlockSpec((BN, 2), lambda i: (i, 0)),
        ],
        out_shape=[
            jax.ShapeDtypeStruct((n, TW), jnp.float32),
            jax.ShapeDtypeStruct((n, 2), jnp.float32),
        ],
    )(acc1, b1r, w2r, a_s2, a_d2)


def _final_body(acc_ref, b2_ref, out_ref):
    a = acc_ref[0, 0] + acc_ref[1, 0]
    out_ref[...] = a[:, 0:128] / (a[:, 128:129] + 1e-16) + b2_ref[0, :][None, :]


def _tc_final(acc2, b2r, n):
    return pl.pallas_call(
        _final_body,
        grid=(n // BN,),
        in_specs=[
            pl.BlockSpec((2, 1, BN, TW), lambda i: (0, 0, i, 0)),
            pl.BlockSpec((1, 128), lambda i: (0, 0)),
        ],
        out_specs=pl.BlockSpec((BN, 128), lambda i: (i, 0)),
        out_shape=jax.ShapeDtypeStruct((n, 128), jnp.float32),
    )(acc2, b2r)


def _make_sc_edge(num_heads, n, e):
    ept = e // (NC * NS)
    sce = 2000
    nsc = ept // sce
    nch = sce // CH
    rpt = 624
    mesh = plsc.VectorSubcoreMesh(core_axis_name="c", subcore_axis_name="s")

    @functools.partial(
        pl.kernel,
        mesh=mesh,
        compiler_params=pltpu.CompilerParams(needs_layout_passes=False,
                                             use_tc_tiling_on_sc=False),
        out_type=jax.ShapeDtypeStruct((NC, num_heads, n, TW), jnp.float32),
        scratch_types=[
            pltpu.VMEM((sce,), jnp.int32),
            pltpu.VMEM((sce,), jnp.int32),
            pltpu.VMEM((CH,), jnp.int32),
            pltpu.VMEM((CH,), jnp.int32),
            pltpu.VMEM((CH,), jnp.int32),
            pltpu.VMEM((CH,), jnp.int32),
            pltpu.VMEM((CH,), jnp.int32),
            pltpu.VMEM((CH,), jnp.int32),
            pltpu.VMEM((CH,), jnp.int32),
            pltpu.VMEM((CH,), jnp.int32),
            pltpu.VMEM((CH,), jnp.int32),
            pltpu.VMEM((CH,), jnp.float32),
            pltpu.VMEM((CH,), jnp.float32),
            pltpu.VMEM((CH,), jnp.float32),
            pltpu.VMEM((CH,), jnp.float32),
            pltpu.VMEM((CH,), jnp.float32),
            pltpu.VMEM((CH,), jnp.float32),
            pltpu.VMEM((CH, TW), jnp.float32),
            pltpu.VMEM((CH, TW), jnp.float32),
            pltpu.VMEM((CH, TW), jnp.float32),
            pltpu.VMEM_SHARED((n, TW), jnp.float32),
            pltpu.SemaphoreType.DMA,
            pltpu.SemaphoreType.DMA,
            pltpu.SemaphoreType.DMA,
            pltpu.SemaphoreType.DMA,
            pltpu.SemaphoreType.DMA,
            pltpu.SemaphoreType.DMA,
        ],
    )
    def sck(tab_hbm, als_hbm, ald_hbm, srce_hbm, dste_hbm, out_hbm,
            src_t, dst_t, gix0, gix1, gix2, dix0, dix1, dix2,
            aix0, aix1, aix2, asb0, asb1, asb2, adb0, adb1, adb2,
            rows0, rows1, rows2,
            acc_sh, gsem0, gsem1, gsem2, ssem0, ssem1, ssem2):
        sid = lax.axis_index("s")
        cid = lax.axis_index("c")
        wid = sid * NC + cid
        base = wid * ept
        z16 = jnp.zeros((16,), jnp.float32)
        buf = ((gix0, dix0, aix0, asb0, adb0, rows0, gsem0, ssem0),
               (gix1, dix1, aix1, asb1, adb1, rows1, gsem1, ssem1),
               (gix2, dix2, aix2, asb2, adb2, rows2, gsem2, ssem2))

        def _fill_zero(k, _):
            i = k // (TW // 16)
            j = k % (TW // 16)
            rows0[i, pl.ds(j * 16, 16)] = z16
            return _

        nz = jnp.where(sid == NS - 1, 40, 39)

        def _clear_acc():
            lax.fori_loop(0, 16 * (TW // 16), _fill_zero, None)

            def _z(z, _):
                pltpu.sync_copy(
                    rows0.at[pl.ds(0, 16), :],
                    acc_sh.at[pl.ds(sid * rpt + z * 16, 16), :])
                return _
            lax.fori_loop(0, nz, _z, None)

        _clear_acc()
        plsc.subcore_barrier()

        def _head(h, _):
            hbase = h * n

            def _do_idx(coff, b):
                gix, dix, aix = buf[b][0], buf[b][1], buf[b][2]

                def _ix(g, _):
                    s16 = src_t[pl.ds(coff + g * 16, 16)]
                    d16 = dst_t[pl.ds(coff + g * 16, 16)]
                    gix[pl.ds(g * 16, 16)] = s16 + hbase
                    dix[pl.ds(g * 16, 16)] = d16
                    aix[pl.ds(g * 16, 16)] = d16 + hbase
                    return _

                lax.fori_loop(0, CH // 16, _ix, None)

            def _issue_gathers(b):
                gix, dix, aix, asb, adb, rows, gsem, _s = buf[b]
                pltpu.async_copy(tab_hbm.at[gix], rows, gsem)
                pltpu.async_copy(als_hbm.at[gix], asb, gsem)
                pltpu.async_copy(ald_hbm.at[aix], adb, gsem)

            def _wait_gathers(b):
                gix, dix, aix, asb, adb, rows, gsem, _s = buf[b]
                pltpu.make_async_copy(tab_hbm.at[gix], rows, gsem).wait()
                pltpu.make_async_copy(als_hbm.at[gix], asb, gsem).wait()
                pltpu.make_async_copy(ald_hbm.at[aix], adb, gsem).wait()

            def _scatter(b):
                rows, ssem = buf[b][5], buf[b][7]
                dix = buf[b][1]
                pltpu.async_copy(rows, acc_sh.at[dix], ssem, add=True)

            def _wait_scatter(b):
                rows, ssem = buf[b][5], buf[b][7]
                dix = buf[b][1]
                pltpu.make_async_copy(rows, acc_sh.at[dix], ssem).wait()

            def _process(b):
                asb, adb, rows = buf[b][3], buf[b][4], buf[b][5]

                def _grp(g, _):
                    ev = asb[pl.ds(g * 16, 16)] + adb[pl.ds(g * 16, 16)]
                    ev = jnp.where(ev > 0, ev, 0.2 * ev)
                    w16 = jnp.exp(ev)
                    for lane in range(16):
                        ws = w16[lane]
                        ei = g * 16 + lane
                        for s in range(TW // 16):
                            rows[ei, pl.ds(s * 16, 16)] = (
                                rows[ei, pl.ds(s * 16, 16)] * ws)
                    return _

                lax.fori_loop(0, CH // 16, _grp, None)

            def _phase(coff, b, guard_first=False):
                bn = (b + 2) % 3
                _wait_gathers(b)
                _process(b)
                _scatter(b)
                if guard_first:
                    pass
                else:
                    _wait_scatter(bn)
                _do_idx(coff + 2 * CH, bn)
                _issue_gathers(bn)

            def _sup(sc_i, _):
                pltpu.sync_copy(
                    srce_hbm.at[pl.ds(base + sc_i * sce, sce)], src_t)
                pltpu.sync_copy(
                    dste_hbm.at[pl.ds(base + sc_i * sce, sce)], dst_t)
                _do_idx(0, 0)
                _issue_gathers(0)
                _do_idx(CH, 1)
                _issue_gathers(1)

                def _triple(i, _):
                    coff = i * 3 * CH

                    _wait_gathers(0)
                    _process(0)
                    _scatter(0)

                    @pl.when(i > 0)
                    def _():
                        _wait_scatter(2)
                    _do_idx(coff + 2 * CH, 2)
                    _issue_gathers(2)
                    _phase(coff + CH, 1)
                    _phase(coff + 2 * CH, 2)
                    return _

                lax.fori_loop(0, (nch - 4) // 3, _triple, None)
                c0 = (nch - 4) // 3 * 3 * CH
                _phase(c0, 0)
                _phase(c0 + CH, 1)
                _wait_gathers(2)
                _process(2)
                _scatter(2)
                _wait_scatter(1)
                _wait_gathers(0)
                _process(0)
                _scatter(0)
                _wait_scatter(2)
                _wait_scatter(0)
                return _

            lax.fori_loop(0, nsc, _sup, None)
            plsc.subcore_barrier()
            pltpu.sync_copy(acc_sh.at[pl.ds(sid * rpt, rpt), :],
                            out_hbm.at[cid, h, pl.ds(sid * rpt, rpt), :])

            @pl.when(sid == NS - 1)
            def _():
                pltpu.sync_copy(
                    acc_sh.at[pl.ds(sid * rpt + rpt, 16), :],
                    out_hbm.at[cid, h, pl.ds(sid * rpt + rpt, 16), :])

            _clear_acc()
            plsc.subcore_barrier()
            return _

        lax.fori_loop(0, num_heads, _head, None)

    return sck


def kernel(x, edge_index, W1, a_src1, a_dst1, b1, W2, a_src2, a_dst2, b2):
    n, f = x.shape
    e = edge_index.shape[1]
    h1 = a_src1.shape[0]

    src = edge_index[0]
    dst = edge_index[1]
    w1r = W1.reshape(f, h1, 128).transpose(1, 0, 2)
    tab1, al1 = _tc_proj1(x, w1r, a_src1.reshape(h1, 1, 128),
                          a_dst1.reshape(h1, 1, 128), n, h1, f)
    acc1 = _make_sc_edge(h1, n, e)(tab1.reshape(h1 * n, TW),
                                   al1[:, 0, :].reshape(h1 * n),
                                   al1[:, 1, :].reshape(h1 * n), src, dst)
    tab2, al2 = _tc_mix(acc1, b1.reshape(h1, 128), W2.reshape(h1, 128, 128),
                        a_src2.reshape(1, 1, 128), a_dst2.reshape(1, 1, 128),
                        n)
    acc2 = _make_sc_edge(1, n, e)(tab2, al2[:, 0], al2[:, 1], src, dst)
    return _tc_final(acc2, b2.reshape(1, 128), n)
